# Initial kernel scaffold; baseline (speedup 1.0000x reference)
import jax
import jax.numpy as jnp
from jax.experimental import pallas as pl


def kernel(x, emb, W1, b1, g1, be1, W2, b2, g2, be2, W3, b3, in_gamma, in_beta):
    raise NotImplementedError("write your pallas kernel here")


# speedup vs baseline: 4.6403x; 4.6403x over previous
import functools

import jax
import jax.numpy as jnp
from jax import lax
from jax.experimental import pallas as pl
from jax.experimental.pallas import tpu as pltpu
from jax.experimental.pallas import tpu_sc as plsc

_B = 16384
_NCAT = 26
_CARD = 100000
_EDIM = 16
_NBIN = 10
_NCONT = 13
_NREST = _NBIN + _NCONT
_GDIM = _NCAT * _EDIM
_H1 = 256
_H2 = 128
_NCLS = 2

_NW = 32
_TOT = _B * _NCAT
_PER_W = _TOT // _NW
_CH = 1024
_NSTEP = _PER_W // _CH
_SUB = _CH // 128


def _gather(idx2d, table):
    mesh = plsc.VectorSubcoreMesh(core_axis_name="c", subcore_axis_name="s")

    @functools.partial(
        pl.kernel,
        mesh=mesh,
        out_type=jax.ShapeDtypeStruct((_TOT, _EDIM), jnp.float32),
        scratch_types=[
            pltpu.VMEM((_SUB, 128), jnp.int32),
            pltpu.VMEM((_CH, _EDIM), jnp.float32),
            pltpu.SemaphoreType.DMA,
        ],
        compiler_params=pltpu.CompilerParams(use_tc_tiling_on_sc=False),
    )
    def gather_k(idx_hbm, table_hbm, out_hbm, idx_v, rows_v, sem):
        wid = lax.axis_index("s") * 2 + lax.axis_index("c")

        def step(t, carry):
            row = wid * (_PER_W // 128) + t * _SUB
            off = wid * _PER_W + t * _CH
            pltpu.sync_copy(idx_hbm.at[pl.ds(row, _SUB)], idx_v)
            copies = [
                pltpu.async_copy(
                    table_hbm.at[idx_v.at[j]],
                    rows_v.at[pl.ds(j * 128, 128)],
                    sem,
                )
                for j in range(_SUB)
            ]
            for c in copies:
                c.wait()
            pltpu.sync_copy(rows_v, out_hbm.at[pl.ds(off, _CH)])
            return carry

        lax.fori_loop(0, _NSTEP, step, 0)

    return gather_k(idx2d, table)


def _ln(h, g, b):
    m = jnp.mean(h, axis=-1, keepdims=True)
    v = jnp.mean((h - m) ** 2, axis=-1, keepdims=True)
    return g * (h - m) / jnp.sqrt(v + 1e-5) + b


def _mlp_body(gath_ref, xr_ref, w1g_ref, w1r_ref, b1_ref, g1_ref, be1_ref,
              w2_ref, b2_ref, g2_ref, be2_ref, w3_ref, b3_ref, igp_ref,
              ibp_ref, o_ref):
    xr = xr_ref[...]
    col = lax.broadcasted_iota(jnp.int32, xr.shape, 1)
    binpart = jnp.clip(jnp.round(xr), 0.0, 1.0)
    contpart = xr * igp_ref[...] + ibp_ref[...]
    rest = jnp.where(col < _NBIN, binpart, contpart)
    z1 = (jnp.dot(gath_ref[...], w1g_ref[...], preferred_element_type=jnp.float32)
          + jnp.dot(rest, w1r_ref[...], preferred_element_type=jnp.float32)
          + b1_ref[...])
    h1 = jnp.maximum(_ln(z1, g1_ref[...], be1_ref[...]), 0.0)
    z2 = jnp.dot(h1, w2_ref[...], preferred_element_type=jnp.float32) + b2_ref[...]
    h2 = jnp.maximum(_ln(z2, g2_ref[...], be2_ref[...]), 0.0)
    o_ref[...] = (jnp.dot(h2, w3_ref[...], preferred_element_type=jnp.float32)
                  + b3_ref[...])


_BB = 512


def _mlp(gath, xr, w1gT, w1rT, b1r, g1r, be1r, w2T, b2r, g2r, be2r, w3T, b3r,
         igp, ibp):
    const = lambda i: (0, 0)
    return pl.pallas_call(
        _mlp_body,
        grid=(_B // _BB,),
        in_specs=[
            pl.BlockSpec((_BB, _GDIM), lambda i: (i, 0)),
            pl.BlockSpec((_BB, _NREST), lambda i: (i, 0)),
            pl.BlockSpec((_GDIM, _H1), const),
            pl.BlockSpec((_NREST, _H1), const),
            pl.BlockSpec((1, _H1), const),
            pl.BlockSpec((1, _H1), const),
            pl.BlockSpec((1, _H1), const),
            pl.BlockSpec((_H1, _H2), const),
            pl.BlockSpec((1, _H2), const),
            pl.BlockSpec((1, _H2), const),
            pl.BlockSpec((1, _H2), const),
            pl.BlockSpec((_H2, _NCLS), const),
            pl.BlockSpec((1, _NCLS), const),
            pl.BlockSpec((1, _NREST), const),
            pl.BlockSpec((1, _NREST), const),
        ],
        out_specs=pl.BlockSpec((_BB, _NCLS), lambda i: (i, 0)),
        out_shape=jax.ShapeDtypeStruct((_B, _NCLS), jnp.float32),
        compiler_params=pltpu.CompilerParams(
            dimension_semantics=("arbitrary",)),
    )(gath, xr, w1gT, w1rT, b1r, g1r, be1r, w2T, b2r, g2r, be2r, w3T, b3r,
      igp, ibp)


def kernel(x, emb, W1, b1, g1, be1, W2, b2, g2, be2, W3, b3, in_gamma,
           in_beta):
    idx = jnp.clip(jnp.round(x[:, :_NCAT]), 0, _CARD - 1).astype(jnp.int32)
    idx = idx + (jnp.arange(_NCAT, dtype=jnp.int32) * _CARD)[None, :]
    idx2d = idx.reshape(_TOT // 128, 128)
    table = emb.reshape(_NCAT * _CARD, _EDIM)
    gath = _gather(idx2d, table).reshape(_B, _GDIM)

    xr = x[:, _NCAT:]
    igp = jnp.concatenate(
        [jnp.ones((_NBIN,), jnp.float32), in_gamma / (1.0 + 1e-6)]
    ).reshape(1, _NREST)
    ibp = jnp.concatenate(
        [jnp.zeros((_NBIN,), jnp.float32), in_beta]
    ).reshape(1, _NREST)

    return _mlp(
        gath, xr,
        W1[:, :_GDIM].T, W1[:, _GDIM:].T,
        b1.reshape(1, _H1), g1.reshape(1, _H1), be1.reshape(1, _H1),
        W2.T, b2.reshape(1, _H2), g2.reshape(1, _H2), be2.reshape(1, _H2),
        W3.T, b3.reshape(1, _NCLS),
        igp, ibp,
    )


# speedup vs baseline: 22.3807x; 4.8231x over previous
import functools

import jax
import jax.numpy as jnp
from jax import lax
from jax.experimental import pallas as pl
from jax.experimental.pallas import tpu as pltpu
from jax.experimental.pallas import tpu_sc as plsc

_B = 16384
_NCAT = 26
_CARD = 100000
_EDIM = 16
_NBIN = 10
_NCONT = 13
_NREST = _NBIN + _NCONT
_GDIM = _NCAT * _EDIM
_H1 = 256
_H2 = 128
_NCLS = 2

_NW = 32
_ROWS_PER_W = _GDIM // _NW
_HALF = _B // 2


def _gather_t(emb_t, idx_t):
    mesh = plsc.VectorSubcoreMesh(core_axis_name="c", subcore_axis_name="s")

    @functools.partial(
        pl.kernel,
        mesh=mesh,
        out_type=jax.ShapeDtypeStruct((_GDIM, _B), jnp.float32),
        scratch_types=[
            pltpu.VMEM((_CARD,), jnp.float32),
            pltpu.VMEM((_HALF,), jnp.int32),
            pltpu.VMEM((_HALF,), jnp.float32),
        ],
        compiler_params=pltpu.CompilerParams(needs_layout_passes=False),
    )
    def gather_k(emb_hbm, idx_hbm, out_hbm, rowbuf, idxbuf, outbuf):
        wid = lax.axis_index("s") * 2 + lax.axis_index("c")

        def do_row(k, carry):
            r = wid * _ROWS_PER_W + k
            c = r // _EDIM
            e = r % _EDIM
            pltpu.sync_copy(emb_hbm.at[c, e, :], rowbuf)

            def do_half(h, carry2):
                pltpu.sync_copy(idx_hbm.at[c, pl.ds(h * _HALF, _HALF)],
                                idxbuf)

                def lp(i, carry3):
                    ii = i * 16
                    iv = idxbuf[pl.ds(ii, 16)]
                    outbuf[pl.ds(ii, 16)] = plsc.load_gather(rowbuf, [iv])
                    return carry3

                lax.fori_loop(0, _HALF // 16, lp, 0)
                pltpu.sync_copy(outbuf,
                                out_hbm.at[r, pl.ds(h * _HALF, _HALF)])
                return carry2

            lax.fori_loop(0, 2, do_half, 0)
            return carry

        lax.fori_loop(0, _ROWS_PER_W, do_row, 0)

    return gather_k(emb_t, idx_t)


def _mlp_body(gath_ref, xr_ref, w1g_ref, w1r_ref, b1_ref, g1_ref, be1_ref,
              w2_ref, b2_ref, g2_ref, be2_ref, w3_ref, b3_ref, igp_ref,
              ibp_ref, o_ref):
    def ln(h, g, b):
        m = jnp.mean(h, axis=0, keepdims=True)
        v = jnp.mean((h - m) ** 2, axis=0, keepdims=True)
        return g * (h - m) / jnp.sqrt(v + 1e-5) + b

    xr = xr_ref[...]
    row = lax.broadcasted_iota(jnp.int32, xr.shape, 0)
    binpart = jnp.clip(jnp.round(xr), 0.0, 1.0)
    contpart = xr * igp_ref[...] + ibp_ref[...]
    rest = jnp.where(row < _NBIN, binpart, contpart)
    z1 = (jnp.dot(w1g_ref[...], gath_ref[...], preferred_element_type=jnp.float32)
          + jnp.dot(w1r_ref[...], rest, preferred_element_type=jnp.float32)
          + b1_ref[...])
    h1 = jnp.maximum(ln(z1, g1_ref[...], be1_ref[...]), 0.0)
    z2 = jnp.dot(w2_ref[...], h1, preferred_element_type=jnp.float32) + b2_ref[...]
    h2 = jnp.maximum(ln(z2, g2_ref[...], be2_ref[...]), 0.0)
    o_ref[...] = (jnp.dot(w3_ref[...], h2, preferred_element_type=jnp.float32)
                  + b3_ref[...])


_BB = 512


def _mlp_t(gath_t, xr_t, w1g, w1r, b1c, g1c, be1c, w2, b2c, g2c, be2c, w3,
           b3c, igc, ibc):
    const = lambda i: (0, 0)
    return pl.pallas_call(
        _mlp_body,
        grid=(_B // _BB,),
        in_specs=[
            pl.BlockSpec((_GDIM, _BB), lambda i: (0, i)),
            pl.BlockSpec((_NREST, _BB), lambda i: (0, i)),
            pl.BlockSpec((_H1, _GDIM), const),
            pl.BlockSpec((_H1, _NREST), const),
            pl.BlockSpec((_H1, 1), const),
            pl.BlockSpec((_H1, 1), const),
            pl.BlockSpec((_H1, 1), const),
            pl.BlockSpec((_H2, _H1), const),
            pl.BlockSpec((_H2, 1), const),
            pl.BlockSpec((_H2, 1), const),
            pl.BlockSpec((_H2, 1), const),
            pl.BlockSpec((_NCLS, _H2), const),
            pl.BlockSpec((_NCLS, 1), const),
            pl.BlockSpec((_NREST, 1), const),
            pl.BlockSpec((_NREST, 1), const),
        ],
        out_specs=pl.BlockSpec((_NCLS, _BB), lambda i: (0, i)),
        out_shape=jax.ShapeDtypeStruct((_NCLS, _B), jnp.float32),
        compiler_params=pltpu.CompilerParams(
            dimension_semantics=("arbitrary",)),
    )(gath_t, xr_t, w1g, w1r, b1c, g1c, be1c, w2, b2c, g2c, be2c, w3, b3c,
      igc, ibc)


def kernel(x, emb, W1, b1, g1, be1, W2, b2, g2, be2, W3, b3, in_gamma,
           in_beta):
    idx_t = jnp.clip(jnp.round(x[:, :_NCAT]), 0, _CARD - 1).astype(
        jnp.int32).T
    emb_t = jnp.transpose(emb, (0, 2, 1))
    gath_t = _gather_t(emb_t, idx_t)

    xr_t = x[:, _NCAT:].T
    igc = jnp.concatenate(
        [jnp.ones((_NBIN,), jnp.float32), in_gamma / (1.0 + 1e-6)]
    ).reshape(_NREST, 1)
    ibc = jnp.concatenate(
        [jnp.zeros((_NBIN,), jnp.float32), in_beta]
    ).reshape(_NREST, 1)

    out_t = _mlp_t(
        gath_t, xr_t,
        W1[:, :_GDIM], W1[:, _GDIM:],
        b1.reshape(_H1, 1), g1.reshape(_H1, 1), be1.reshape(_H1, 1),
        W2, b2.reshape(_H2, 1), g2.reshape(_H2, 1), be2.reshape(_H2, 1),
        W3, b3.reshape(_NCLS, 1),
        igc, ibc,
    )
    return out_t.T


# speedup vs baseline: 23.7932x; 1.0631x over previous
import functools

import jax
import jax.numpy as jnp
from jax import lax
from jax.experimental import pallas as pl
from jax.experimental.pallas import tpu as pltpu
from jax.experimental.pallas import tpu_sc as plsc

_B = 16384
_NCAT = 26
_CARD = 100000
_EDIM = 16
_NBIN = 10
_NCONT = 13
_NREST = _NBIN + _NCONT
_GDIM = _NCAT * _EDIM
_H1 = 256
_H2 = 128
_NCLS = 2

_NW = 32
_ROWS_PER_W = _GDIM // _NW
_Q = 4096


def _gather_t(emb_t, idx_t):
    mesh = plsc.VectorSubcoreMesh(core_axis_name="c", subcore_axis_name="s")

    @functools.partial(
        pl.kernel,
        mesh=mesh,
        out_type=jax.ShapeDtypeStruct((_GDIM, _B), jnp.float32),
        scratch_types=[
            pltpu.VMEM((_CARD,), jnp.float32),
            pltpu.VMEM((_B,), jnp.int32),
            pltpu.VMEM((_Q,), jnp.float32),
            pltpu.VMEM((_Q,), jnp.float32),
            pltpu.SemaphoreType.DMA,
            pltpu.SemaphoreType.DMA,
        ],
        compiler_params=pltpu.CompilerParams(needs_layout_passes=False),
    )
    def gather_k(emb_hbm, idx_hbm, out_hbm, rowbuf, idxbuf, outq0, outq1,
                 sem_row, sem_w):
        wid = lax.axis_index("s") * 2 + lax.axis_index("c")

        def do_row(k, cprev):
            r = wid * _ROWS_PER_W + k
            c = r // _EDIM
            e = r % _EDIM
            rcp = pltpu.async_copy(emb_hbm.at[c, e, :], rowbuf, sem_row)

            @pl.when(c != cprev)
            def _():
                pltpu.sync_copy(idx_hbm.at[c, :], idxbuf)

            rcp.wait()
            handles = []
            for h in range(_B // _Q):
                ob = outq0 if h % 2 == 0 else outq1
                if h >= 2:
                    handles[h - 2].wait()

                def lp(i, cc, h=h, ob=ob):
                    iv = idxbuf[pl.ds(h * _Q + i * 16, 16)]
                    ob[pl.ds(i * 16, 16)] = plsc.load_gather(rowbuf, [iv])
                    return cc

                lax.fori_loop(0, _Q // 16, lp, 0)
                handles.append(
                    pltpu.async_copy(ob, out_hbm.at[r, pl.ds(h * _Q, _Q)],
                                     sem_w))
            handles[-2].wait()
            handles[-1].wait()
            return c

        lax.fori_loop(0, _ROWS_PER_W, do_row, -1)

    return gather_k(emb_t, idx_t)


def _mlp_body(gath_ref, xr_ref, w1g_ref, w1r_ref, b1_ref, g1_ref, be1_ref,
              w2_ref, b2_ref, g2_ref, be2_ref, w3_ref, b3_ref, igp_ref,
              ibp_ref, o_ref):
    def ln(h, g, b):
        m = jnp.mean(h, axis=0, keepdims=True)
        v = jnp.mean((h - m) ** 2, axis=0, keepdims=True)
        return g * (h - m) / jnp.sqrt(v + 1e-5) + b

    xr = xr_ref[...]
    row = lax.broadcasted_iota(jnp.int32, xr.shape, 0)
    binpart = jnp.clip(jnp.round(xr), 0.0, 1.0)
    contpart = xr * igp_ref[...] + ibp_ref[...]
    rest = jnp.where(row < _NBIN, binpart, contpart)
    z1 = (jnp.dot(w1g_ref[...], gath_ref[...], preferred_element_type=jnp.float32)
          + jnp.dot(w1r_ref[...], rest, preferred_element_type=jnp.float32)
          + b1_ref[...])
    h1 = jnp.maximum(ln(z1, g1_ref[...], be1_ref[...]), 0.0)
    z2 = jnp.dot(w2_ref[...], h1, preferred_element_type=jnp.float32) + b2_ref[...]
    h2 = jnp.maximum(ln(z2, g2_ref[...], be2_ref[...]), 0.0)
    o_ref[...] = (jnp.dot(w3_ref[...], h2, preferred_element_type=jnp.float32)
                  + b3_ref[...])


_BB = 1024


def _mlp_t(gath_t, xr_t, w1g, w1r, b1c, g1c, be1c, w2, b2c, g2c, be2c, w3,
           b3c, igc, ibc):
    const = lambda i: (0, 0)
    return pl.pallas_call(
        _mlp_body,
        grid=(_B // _BB,),
        in_specs=[
            pl.BlockSpec((_GDIM, _BB), lambda i: (0, i)),
            pl.BlockSpec((_NREST, _BB), lambda i: (0, i)),
            pl.BlockSpec((_H1, _GDIM), const),
            pl.BlockSpec((_H1, _NREST), const),
            pl.BlockSpec((_H1, 1), const),
            pl.BlockSpec((_H1, 1), const),
            pl.BlockSpec((_H1, 1), const),
            pl.BlockSpec((_H2, _H1), const),
            pl.BlockSpec((_H2, 1), const),
            pl.BlockSpec((_H2, 1), const),
            pl.BlockSpec((_H2, 1), const),
            pl.BlockSpec((_NCLS, _H2), const),
            pl.BlockSpec((_NCLS, 1), const),
            pl.BlockSpec((_NREST, 1), const),
            pl.BlockSpec((_NREST, 1), const),
        ],
        out_specs=pl.BlockSpec((_NCLS, _BB), lambda i: (0, i)),
        out_shape=jax.ShapeDtypeStruct((_NCLS, _B), jnp.float32),
        compiler_params=pltpu.CompilerParams(
            dimension_semantics=("arbitrary",)),
    )(gath_t, xr_t, w1g, w1r, b1c, g1c, be1c, w2, b2c, g2c, be2c, w3, b3c,
      igc, ibc)


def kernel(x, emb, W1, b1, g1, be1, W2, b2, g2, be2, W3, b3, in_gamma,
           in_beta):
    idx_t = jnp.clip(jnp.round(x[:, :_NCAT]), 0, _CARD - 1).astype(
        jnp.int32).T
    emb_t = jnp.transpose(emb, (0, 2, 1))
    gath_t = _gather_t(emb_t, idx_t)

    xr_t = x[:, _NCAT:].T
    igc = jnp.concatenate(
        [jnp.ones((_NBIN,), jnp.float32), in_gamma / (1.0 + 1e-6)]
    ).reshape(_NREST, 1)
    ibc = jnp.concatenate(
        [jnp.zeros((_NBIN,), jnp.float32), in_beta]
    ).reshape(_NREST, 1)

    out_t = _mlp_t(
        gath_t, xr_t,
        W1[:, :_GDIM], W1[:, _GDIM:],
        b1.reshape(_H1, 1), g1.reshape(_H1, 1), be1.reshape(_H1, 1),
        W2, b2.reshape(_H2, 1), g2.reshape(_H2, 1), be2.reshape(_H2, 1),
        W3, b3.reshape(_NCLS, 1),
        igc, ibc,
    )
    return out_t.T


# speedup vs baseline: 40.0650x; 1.6839x over previous
import functools

import jax
import jax.numpy as jnp
from jax import lax
from jax.experimental import pallas as pl
from jax.experimental.pallas import tpu as pltpu
from jax.experimental.pallas import tpu_sc as plsc

_B = 16384
_NCAT = 26
_CARD = 100000
_EDIM = 16
_NBIN = 10
_NCONT = 13
_NREST = _NBIN + _NCONT
_GDIM = _NCAT * _EDIM
_H1 = 256
_H2 = 128
_NCLS = 2

_NW = 32
_ROWS_PER_W = _GDIM // _NW
_Q = 4096


def _gather_t(emb_t, idx_t):
    mesh = plsc.VectorSubcoreMesh(core_axis_name="c", subcore_axis_name="s")

    @functools.partial(
        pl.kernel,
        mesh=mesh,
        out_type=jax.ShapeDtypeStruct((_GDIM, _B), jnp.float32),
        scratch_types=[
            pltpu.VMEM((_CARD,), jnp.float32),
            pltpu.VMEM((_B,), jnp.int32),
            pltpu.VMEM((_Q,), jnp.float32),
            pltpu.VMEM((_Q,), jnp.float32),
            pltpu.SemaphoreType.DMA,
            pltpu.SemaphoreType.DMA,
        ],
        compiler_params=pltpu.CompilerParams(needs_layout_passes=False),
    )
    def gather_k(emb_hbm, idx_hbm, out_hbm, rowbuf, idxbuf, outq0, outq1,
                 sem_row, sem_w):
        wid = lax.axis_index("s") * 2 + lax.axis_index("c")

        def do_row(k, cprev):
            r = wid * _ROWS_PER_W + k
            c = r // _EDIM
            e = r % _EDIM
            rcp = pltpu.async_copy(emb_hbm.at[c, e, :], rowbuf, sem_row)

            @pl.when(c != cprev)
            def _():
                pltpu.sync_copy(idx_hbm.at[c, :], idxbuf)

            rcp.wait()
            handles = []
            for h in range(_B // _Q):
                ob = outq0 if h % 2 == 0 else outq1
                if h >= 2:
                    handles[h - 2].wait()

                def gather_quarter(hh, obuf):
                    @plsc.parallel_loop(0, _Q, 16, unroll=8)
                    def _(ii):
                        iv = idxbuf[pl.ds(hh * _Q + ii, 16)]
                        obuf[pl.ds(ii, 16)] = plsc.load_gather(rowbuf, [iv])

                gather_quarter(h, ob)
                handles.append(
                    pltpu.async_copy(ob, out_hbm.at[r, pl.ds(h * _Q, _Q)],
                                     sem_w))
            handles[-2].wait()
            handles[-1].wait()
            return c

        lax.fori_loop(0, _ROWS_PER_W, do_row, -1)

    return gather_k(emb_t, idx_t)


def _mlp_body(gath_ref, xr_ref, w1g_ref, w1r_ref, b1_ref, g1_ref, be1_ref,
              w2_ref, b2_ref, g2_ref, be2_ref, w3_ref, b3_ref, igp_ref,
              ibp_ref, o_ref):
    def ln(h, g, b):
        m = jnp.mean(h, axis=0, keepdims=True)
        v = jnp.mean((h - m) ** 2, axis=0, keepdims=True)
        return g * (h - m) / jnp.sqrt(v + 1e-5) + b

    xr = xr_ref[...]
    row = lax.broadcasted_iota(jnp.int32, xr.shape, 0)
    binpart = jnp.clip(jnp.round(xr), 0.0, 1.0)
    contpart = xr * igp_ref[...] + ibp_ref[...]
    rest = jnp.where(row < _NBIN, binpart, contpart)
    z1 = (jnp.dot(w1g_ref[...], gath_ref[...], preferred_element_type=jnp.float32)
          + jnp.dot(w1r_ref[...], rest, preferred_element_type=jnp.float32)
          + b1_ref[...])
    h1 = jnp.maximum(ln(z1, g1_ref[...], be1_ref[...]), 0.0)
    z2 = jnp.dot(w2_ref[...], h1, preferred_element_type=jnp.float32) + b2_ref[...]
    h2 = jnp.maximum(ln(z2, g2_ref[...], be2_ref[...]), 0.0)
    o_ref[...] = (jnp.dot(w3_ref[...], h2, preferred_element_type=jnp.float32)
                  + b3_ref[...])


_BB = 1024


def _mlp_t(gath_t, xr_t, w1g, w1r, b1c, g1c, be1c, w2, b2c, g2c, be2c, w3,
           b3c, igc, ibc):
    const = lambda i: (0, 0)
    return pl.pallas_call(
        _mlp_body,
        grid=(_B // _BB,),
        in_specs=[
            pl.BlockSpec((_GDIM, _BB), lambda i: (0, i)),
            pl.BlockSpec((_NREST, _BB), lambda i: (0, i)),
            pl.BlockSpec((_H1, _GDIM), const),
            pl.BlockSpec((_H1, _NREST), const),
            pl.BlockSpec((_H1, 1), const),
            pl.BlockSpec((_H1, 1), const),
            pl.BlockSpec((_H1, 1), const),
            pl.BlockSpec((_H2, _H1), const),
            pl.BlockSpec((_H2, 1), const),
            pl.BlockSpec((_H2, 1), const),
            pl.BlockSpec((_H2, 1), const),
            pl.BlockSpec((_NCLS, _H2), const),
            pl.BlockSpec((_NCLS, 1), const),
            pl.BlockSpec((_NREST, 1), const),
            pl.BlockSpec((_NREST, 1), const),
        ],
        out_specs=pl.BlockSpec((_NCLS, _BB), lambda i: (0, i)),
        out_shape=jax.ShapeDtypeStruct((_NCLS, _B), jnp.float32),
        compiler_params=pltpu.CompilerParams(
            dimension_semantics=("arbitrary",)),
    )(gath_t, xr_t, w1g, w1r, b1c, g1c, be1c, w2, b2c, g2c, be2c, w3, b3c,
      igc, ibc)


def kernel(x, emb, W1, b1, g1, be1, W2, b2, g2, be2, W3, b3, in_gamma,
           in_beta):
    idx_t = jnp.clip(jnp.round(x[:, :_NCAT]), 0, _CARD - 1).astype(
        jnp.int32).T
    emb_t = jnp.transpose(emb, (0, 2, 1))
    gath_t = _gather_t(emb_t, idx_t)

    xr_t = x[:, _NCAT:].T
    igc = jnp.concatenate(
        [jnp.ones((_NBIN,), jnp.float32), in_gamma / (1.0 + 1e-6)]
    ).reshape(_NREST, 1)
    ibc = jnp.concatenate(
        [jnp.zeros((_NBIN,), jnp.float32), in_beta]
    ).reshape(_NREST, 1)

    out_t = _mlp_t(
        gath_t, xr_t,
        W1[:, :_GDIM], W1[:, _GDIM:],
        b1.reshape(_H1, 1), g1.reshape(_H1, 1), be1.reshape(_H1, 1),
        W2, b2.reshape(_H2, 1), g2.reshape(_H2, 1), be2.reshape(_H2, 1),
        W3, b3.reshape(_NCLS, 1),
        igc, ibc,
    )
    return out_t.T


# speedup vs baseline: 41.4119x; 1.0336x over previous
import functools

import jax
import jax.numpy as jnp
from jax import lax
from jax.experimental import pallas as pl
from jax.experimental.pallas import tpu as pltpu
from jax.experimental.pallas import tpu_sc as plsc

_B = 16384
_NCAT = 26
_CARD = 100000
_EDIM = 16
_NBIN = 10
_NCONT = 13
_NREST = _NBIN + _NCONT
_GDIM = _NCAT * _EDIM
_H1 = 256
_H2 = 128
_NCLS = 2

_NW = 32
_ROWS_PER_W = _GDIM // _NW
_Q = 4096


def _gather_t(emb_t, idx_t):
    mesh = plsc.VectorSubcoreMesh(core_axis_name="c", subcore_axis_name="s")

    @functools.partial(
        pl.kernel,
        mesh=mesh,
        out_type=jax.ShapeDtypeStruct((_GDIM, _B), jnp.float32),
        scratch_types=[
            pltpu.VMEM((_CARD,), jnp.float32),
            pltpu.VMEM((_B,), jnp.int32),
            pltpu.VMEM((_Q,), jnp.float32),
            pltpu.VMEM((_Q,), jnp.float32),
            pltpu.SemaphoreType.DMA,
            pltpu.SemaphoreType.DMA,
        ],
        compiler_params=pltpu.CompilerParams(needs_layout_passes=False),
    )
    def gather_k(emb_hbm, idx_hbm, out_hbm, rowbuf, idxbuf, outq0, outq1,
                 sem_row, sem_w):
        wid = lax.axis_index("s") * 2 + lax.axis_index("c")

        def do_row(k, cprev):
            r = wid * _ROWS_PER_W + k
            c = r // _EDIM
            e = r % _EDIM
            rcp = pltpu.async_copy(emb_hbm.at[c, e, :], rowbuf, sem_row)

            @pl.when(c != cprev)
            def _():
                pltpu.sync_copy(idx_hbm.at[c, :], idxbuf)

            rcp.wait()
            handles = []
            for h in range(_B // _Q):
                ob = outq0 if h % 2 == 0 else outq1
                if h >= 2:
                    handles[h - 2].wait()

                def gather_quarter(hh, obuf):
                    @plsc.parallel_loop(0, _Q, 16, unroll=8)
                    def _(ii):
                        iv = idxbuf[pl.ds(hh * _Q + ii, 16)]
                        obuf[pl.ds(ii, 16)] = plsc.load_gather(rowbuf, [iv])

                gather_quarter(h, ob)
                handles.append(
                    pltpu.async_copy(ob, out_hbm.at[r, pl.ds(h * _Q, _Q)],
                                     sem_w))
            handles[-2].wait()
            handles[-1].wait()
            return c

        lax.fori_loop(0, _ROWS_PER_W, do_row, -1)

    return gather_k(emb_t, idx_t)


def _mlp_body(gath_ref, xr_ref, w1g_ref, w1r_ref, b1_ref, g1_ref, be1_ref,
              w2_ref, b2_ref, g2_ref, be2_ref, w3_ref, b3_ref, igp_ref,
              ibp_ref, o_ref):
    def ln(h, g, b):
        m = jnp.mean(h, axis=0, keepdims=True)
        v = jnp.mean((h - m) ** 2, axis=0, keepdims=True)
        return g * (h - m) / jnp.sqrt(v + 1e-5) + b

    xr = xr_ref[...]
    row = lax.broadcasted_iota(jnp.int32, xr.shape, 0)
    binpart = jnp.clip(jnp.round(xr), 0.0, 1.0)
    contpart = xr * igp_ref[...] + ibp_ref[...]
    rest = jnp.where(row < _NBIN, binpart, contpart)
    z1 = (jnp.dot(w1g_ref[...], gath_ref[...], preferred_element_type=jnp.float32)
          + jnp.dot(w1r_ref[...], rest, preferred_element_type=jnp.float32)
          + b1_ref[...])
    h1 = jnp.maximum(ln(z1, g1_ref[...], be1_ref[...]), 0.0)
    z2 = jnp.dot(w2_ref[...], h1, preferred_element_type=jnp.float32) + b2_ref[...]
    h2 = jnp.maximum(ln(z2, g2_ref[...], be2_ref[...]), 0.0)
    o_ref[...] = (jnp.dot(w3_ref[...], h2, preferred_element_type=jnp.float32)
                  + b3_ref[...])


_BB = 2048


def _mlp_t(gath_t, xr_t, w1g, w1r, b1c, g1c, be1c, w2, b2c, g2c, be2c, w3,
           b3c, igc, ibc):
    const = lambda i: (0, 0)
    return pl.pallas_call(
        _mlp_body,
        grid=(_B // _BB,),
        in_specs=[
            pl.BlockSpec((_GDIM, _BB), lambda i: (0, i)),
            pl.BlockSpec((_NREST, _BB), lambda i: (0, i)),
            pl.BlockSpec((_H1, _GDIM), const),
            pl.BlockSpec((_H1, _NREST), const),
            pl.BlockSpec((_H1, 1), const),
            pl.BlockSpec((_H1, 1), const),
            pl.BlockSpec((_H1, 1), const),
            pl.BlockSpec((_H2, _H1), const),
            pl.BlockSpec((_H2, 1), const),
            pl.BlockSpec((_H2, 1), const),
            pl.BlockSpec((_H2, 1), const),
            pl.BlockSpec((_NCLS, _H2), const),
            pl.BlockSpec((_NCLS, 1), const),
            pl.BlockSpec((_NREST, 1), const),
            pl.BlockSpec((_NREST, 1), const),
        ],
        out_specs=pl.BlockSpec((_NCLS, _BB), lambda i: (0, i)),
        out_shape=jax.ShapeDtypeStruct((_NCLS, _B), jnp.float32),
        compiler_params=pltpu.CompilerParams(
            dimension_semantics=("arbitrary",)),
    )(gath_t, xr_t, w1g, w1r, b1c, g1c, be1c, w2, b2c, g2c, be2c, w3, b3c,
      igc, ibc)


def kernel(x, emb, W1, b1, g1, be1, W2, b2, g2, be2, W3, b3, in_gamma,
           in_beta):
    idx_t = jnp.clip(jnp.round(x[:, :_NCAT]), 0, _CARD - 1).astype(
        jnp.int32).T
    emb_t = jnp.transpose(emb, (0, 2, 1))
    gath_t = _gather_t(emb_t, idx_t)

    xr_t = x[:, _NCAT:].T
    igc = jnp.concatenate(
        [jnp.ones((_NBIN,), jnp.float32), in_gamma / (1.0 + 1e-6)]
    ).reshape(_NREST, 1)
    ibc = jnp.concatenate(
        [jnp.zeros((_NBIN,), jnp.float32), in_beta]
    ).reshape(_NREST, 1)

    out_t = _mlp_t(
        gath_t, xr_t,
        W1[:, :_GDIM], W1[:, _GDIM:],
        b1.reshape(_H1, 1), g1.reshape(_H1, 1), be1.reshape(_H1, 1),
        W2, b2.reshape(_H2, 1), g2.reshape(_H2, 1), be2.reshape(_H2, 1),
        W3, b3.reshape(_NCLS, 1),
        igc, ibc,
    )
    return out_t.T
